# baseline (speedup 1.0000x reference)
import functools

import jax
import jax.numpy as jnp
from jax.experimental import pallas as pl
from jax.experimental.pallas import tpu as pltpu


def _mm_kernel(x_ref, w_ref, o_ref):
    o_ref[...] = jnp.dot(x_ref[...], w_ref[...],
                         preferred_element_type=jnp.float32)


def _mm(x, w, bm=512):
    m, k = x.shape
    _, n = w.shape
    mp = ((m + bm - 1) // bm) * bm
    if mp != m:
        x = jnp.pad(x, ((0, mp - m), (0, 0)))
    grid = (mp // bm,)
    out = pl.pallas_call(
        _mm_kernel,
        grid=grid,
        in_specs=[
            pl.BlockSpec((bm, k), lambda i: (i, 0)),
            pl.BlockSpec((k, n), lambda i: (0, 0)),
        ],
        out_specs=pl.BlockSpec((bm, n), lambda i: (i, 0)),
        out_shape=jax.ShapeDtypeStruct((mp, n), jnp.float32),
    )(x, w)
    return out[:m] if mp != m else out


def _gat_layer(xin, ei, W, a_s, a_d, b, Wsk, bsk, g, be, H, C, concat, ns):
    xt = xin[:ns]
    h2 = xt @ W
    sk = xt @ Wsk + bsk
    h = h2.reshape(ns, H, C)
    asrc = (h * a_s[None, :, :]).sum(-1)
    adst = (h * a_d[None, :, :]).sum(-1)
    loops = jnp.arange(ns, dtype=ei.dtype)
    es = jnp.concatenate([ei[0], loops])
    ed = jnp.concatenate([ei[1], loops])
    alpha = jax.nn.leaky_relu(asrc[es] + adst[ed], 0.2)
    amax = jax.ops.segment_max(alpha, ed, num_segments=ns)
    ex = jnp.exp(alpha - amax[ed])
    den = jax.ops.segment_sum(ex, ed, num_segments=ns)
    w = ex / (den[ed] + 1e-16)
    out = jax.ops.segment_sum(h[es] * w[:, :, None], ed, num_segments=ns)
    if concat:
        out = out.reshape(ns, H * C)
    else:
        out = out.mean(axis=1)
    hfin = out + b + sk
    m = hfin.mean(axis=0)
    v = hfin.var(axis=0)
    hfin = g * (hfin - m) / jnp.sqrt(v + 1e-5) + be
    return jax.nn.elu(hfin)


def kernel(x, edge_index_0, edge_index_1, edge_index_2, size_0, size_1,
           size_2, W0, a_src0, a_dst0, b0, Wsk0, bsk0, g0, be0, W1, a_src1,
           a_dst1, b1, Wsk1, bsk1, g1, be1, W2, a_src2, a_dst2, b2, Wsk2,
           bsk2, g2, be2):
    h = _gat_layer(x, edge_index_0, W0, a_src0, a_dst0, b0, Wsk0, bsk0,
                   g0, be0, 4, 64, True, 40000)
    h = _gat_layer(h, edge_index_1, W1, a_src1, a_dst1, b1, Wsk1, bsk1,
                   g1, be1, 4, 64, True, 10000)
    h = _gat_layer(h, edge_index_2, W2, a_src2, a_dst2, b2, Wsk2, bsk2,
                   g2, be2, 4, 128, False, 2500)
    return h


# speedup vs baseline: 5.8184x; 5.8184x over previous
import functools

import jax
import jax.numpy as jnp
from jax import lax
from jax.experimental import pallas as pl
from jax.experimental.pallas import tpu as pltpu
from jax.experimental.pallas import tpu_sc as plsc

NC, NS = 2, 16
NW = NC * NS
KA = 400
KB = 32
ZR = 16
CHS = 12
CHM = 4095

_SC_PARAMS = pltpu.CompilerParams(use_tc_tiling_on_sc=False,
                                  needs_layout_passes=False)


def _pre_body(x_ref, w_ref, wsk_ref, aall_ref, bsk_ref,
              h2_ref, sk_ref, aa_ref, exs_ref):
    h2 = jnp.dot(x_ref[...], w_ref[...], preferred_element_type=jnp.float32)
    h2_ref[...] = h2
    sk_ref[...] = jnp.dot(x_ref[...], wsk_ref[...],
                          preferred_element_type=jnp.float32) + bsk_ref[...]
    aa = jnp.dot(h2, aall_ref[...], preferred_element_type=jnp.float32)
    aa_ref[...] = aa
    s = aa[:, :4] + aa[:, 4:]
    exs_ref[...] = jnp.concatenate(
        [jnp.exp(jnp.maximum(s, 0.2 * s)), jnp.zeros_like(s)], axis=1)


def _pre(xp, W, Wsk, Aall, bsk, NP, HC, dout, bm=512):
    din = xp.shape[1]
    return pl.pallas_call(
        _pre_body,
        grid=(NP // bm,),
        in_specs=[
            pl.BlockSpec((bm, din), lambda i: (i, 0)),
            pl.BlockSpec((din, HC), lambda i: (0, 0)),
            pl.BlockSpec((din, dout), lambda i: (0, 0)),
            pl.BlockSpec((HC, 8), lambda i: (0, 0)),
            pl.BlockSpec((1, dout), lambda i: (0, 0)),
        ],
        out_specs=[
            pl.BlockSpec((bm, HC), lambda i: (i, 0)),
            pl.BlockSpec((bm, dout), lambda i: (i, 0)),
            pl.BlockSpec((bm, 8), lambda i: (i, 0)),
            pl.BlockSpec((bm, 8), lambda i: (i, 0)),
        ],
        out_shape=[
            jax.ShapeDtypeStruct((NP, HC), jnp.float32),
            jax.ShapeDtypeStruct((NP, dout), jnp.float32),
            jax.ShapeDtypeStruct((NP, 8), jnp.float32),
            jax.ShapeDtypeStruct((NP, 8), jnp.float32),
        ],
    )(xp, W, Wsk, Aall, bsk)


def _phase_a(es, ed, aa, exs, NP):
    E = es.shape[0]
    nblocks = E // KA
    rpt = NP // NS
    mesh = plsc.VectorSubcoreMesh(core_axis_name="c", subcore_axis_name="s")

    @functools.partial(
        pl.kernel,
        out_type=[
            jax.ShapeDtypeStruct((E, 8), jnp.float32),
            jax.ShapeDtypeStruct((NC, NP, 8), jnp.float32),
        ],
        mesh=mesh,
        compiler_params=_SC_PARAMS,
        scratch_types=[
            pltpu.VMEM((KA,), jnp.int32),
            pltpu.VMEM((KA,), jnp.int32),
            pltpu.VMEM((KA, 8), jnp.float32),
            pltpu.VMEM((KA, 8), jnp.float32),
            pltpu.VMEM((KA, 8), jnp.float32),
            pltpu.VMEM((32, 8), jnp.float32),
            pltpu.VMEM_SHARED((NP, 8), jnp.float32),
            pltpu.SemaphoreType.DMA,
            pltpu.SemaphoreType.DMA,
        ],
    )
    def k(es_h, ed_h, aa_h, exs_h, ex_h, denp_h,
          esb, edb, ab, bb, exb, zb, den_sp, sem1, sem2):
        cid = lax.axis_index("c")
        sid = lax.axis_index("s")
        wid = sid * NC + cid
        r0 = sid * rpt
        iota = lax.iota(jnp.int32, 16)
        zf = jnp.zeros((16,), jnp.float32)

        @pl.when(cid == 0)
        def _():
            pltpu.sync_copy(exs_h.at[pl.ds(r0, rpt)],
                            den_sp.at[pl.ds(r0, rpt)])

        @pl.when(cid == 1)
        def _():
            for i in range(16):
                plsc.store_scatter(zb, [(i * 16 + iota) >> 3, iota & 7], zf)

            @pl.loop(0, rpt // 32)
            def _(i):
                pltpu.sync_copy(zb, den_sp.at[pl.ds(r0 + i * 32, 32)])

        plsc.subcore_barrier()

        @pl.loop(wid, nblocks, step=NW)
        def _(b):
            e0 = b * KA
            pltpu.sync_copy(es_h.at[pl.ds(e0, KA)], esb)
            pltpu.sync_copy(ed_h.at[pl.ds(e0, KA)], edb)
            c1 = pltpu.async_copy(aa_h.at[esb], ab, sem1)
            c2 = pltpu.async_copy(aa_h.at[edb], bb, sem2)
            c1.wait()
            c2.wait()
            for g in range(KA * 8 // 16):
                ridx = g * 2 + (iota >> 3)
                cdx = iota & 7
                hi = cdx > 3
                v = plsc.load_gather(ab, [ridx, jnp.where(hi, 0, cdx)]) + \
                    plsc.load_gather(bb, [ridx, jnp.where(hi, 0, cdx + 4)])
                v = jnp.exp(jnp.maximum(v, 0.2 * v))
                v = jnp.where(hi, 0.0, v)
                plsc.store_scatter(exb, [ridx, cdx], v)
            pltpu.sync_copy(exb, ex_h.at[pl.ds(e0, KA)])
            pltpu.sync_copy(exb, den_sp.at[edb], add=True)

        plsc.subcore_barrier()
        pltpu.sync_copy(den_sp.at[pl.ds(r0, rpt)],
                        denp_h.at[cid].at[pl.ds(r0, rpt)])

    return k(es, ed, aa, exs)


def _phase_b(es, ed, ex, den0, den1, h2, NP, HC, CH, NCHK, EB):
    E = es.shape[0]
    nblocks = E // KA
    rpa = CH // NS
    sh = 8 if HC == 256 else 9
    mesh = plsc.VectorSubcoreMesh(core_axis_name="c", subcore_axis_name="s")

    @functools.partial(
        pl.kernel,
        out_type=jax.ShapeDtypeStruct((NC, NCHK * CH, HC), jnp.float32),
        mesh=mesh,
        compiler_params=_SC_PARAMS,
        scratch_types=[
            pltpu.VMEM((KA,), jnp.int32),
            pltpu.VMEM((KA,), jnp.int32),
            pltpu.VMEM((EB,), jnp.int32),
            pltpu.VMEM((EB,), jnp.int32),
            pltpu.VMEM((EB,), jnp.int32),
            pltpu.VMEM((KB + 16, HC), jnp.float32),
            pltpu.VMEM((8, HC), jnp.float32),
            pltpu.VMEM((KB + 16, 8), jnp.float32),
            pltpu.VMEM((KB + 16, 8), jnp.float32),
            pltpu.VMEM((KB + 16, 8), jnp.float32),
            pltpu.VMEM(((KB + 16) * 8,), jnp.float32),
            pltpu.VMEM((KB + 16,), jnp.int32),
            pltpu.VMEM((KB + 16,), jnp.int32),
            pltpu.VMEM((KB + 16,), jnp.int32),
            pltpu.VMEM((KB + 16,), jnp.int32),
            pltpu.SMEM((16,), jnp.int32),
            pltpu.SMEM((16,), jnp.int32),
            pltpu.VMEM_SHARED((CH + 8, HC), jnp.float32),
            pltpu.SemaphoreType.DMA,
            pltpu.SemaphoreType.DMA,
            pltpu.SemaphoreType.DMA,
            pltpu.SemaphoreType.DMA,
        ],
    )
    def k(es_h, ed_h, ex_h, d0_h, d1_h, h2_h, outp_h,
          esb, edb, bes, bdl, bei, hbuf, zbuf, exg, d0g, d1g, wbuf,
          idxw, idxd, idxe, idxf, ptrs, aoffs, acc_sp,
          sem1, sem2, sem3, sem4):
        cid = lax.axis_index("c")
        sid = lax.axis_index("s")
        wid = sid * NC + cid
        iota = lax.iota(jnp.int32, 16)
        zf = jnp.zeros((16,), jnp.float32)
        zi = jnp.zeros((16,), jnp.int32)

        @pl.loop(0, 8 * HC // 16)
        def _(i):
            flat = i * 16 + iota
            plsc.store_scatter(zbuf, [flat >> sh, flat & (HC - 1)], zf)

        @pl.loop(0, EB // 16)
        def _(i):
            bes[pl.ds(i * 16, 16)] = zi
            bdl[pl.ds(i * 16, 16)] = zi
            bei[pl.ds(i * 16, 16)] = zi

        @pl.loop(wid, nblocks, step=NW, init_carry=jnp.zeros((16,),
                                                             jnp.int32))
        def counts(b, cc):
            pltpu.sync_copy(ed_h.at[pl.ds(b * KA, KA)], edb)
            for g in range(KA // 16):
                ch = edb[pl.ds(g * 16, 16)] >> CHS
                for c in range(NCHK):
                    pc = jnp.sum(jnp.where(ch == c, 1, 0).astype(jnp.int32))
                    cc = cc + jnp.where(iota == c, pc, 0)
            return cc

        acnt = (counts + 15) & ~15
        excl = plsc.cumsum(acnt) - acnt
        for c in range(NCHK):
            off = jnp.sum(jnp.where(iota == c, excl, 0))
            aoffs[c] = off
            ptrs[c] = off

        @pl.loop(wid, nblocks, step=NW)
        def _(b):
            e0 = b * KA
            pltpu.sync_copy(es_h.at[pl.ds(e0, KA)], esb)
            pltpu.sync_copy(ed_h.at[pl.ds(e0, KA)], edb)

            @pl.loop(0, NCHK)
            def _(c):
                p = ptrs[c]
                for g in range(KA // 16):
                    ev = edb[pl.ds(g * 16, 16)]
                    esv = esb[pl.ds(g * 16, 16)]
                    ch = ev >> CHS
                    m = ch == c
                    mi = jnp.where(m, 1, 0).astype(jnp.int32)
                    rank = plsc.cumsum(mi) - mi
                    dest = p + rank
                    plsc.store_scatter(bes, [dest], esv, mask=m)
                    plsc.store_scatter(bdl, [dest], ev & CHM, mask=m)
                    plsc.store_scatter(bei, [dest], e0 + g * 16 + iota,
                                       mask=m)
                    p = p + jnp.sum(mi)
                ptrs[c] = p

        @pl.loop(0, NCHK)
        def _(c):
            for q in range(rpa // 8):
                pltpu.sync_copy(
                    zbuf, acc_sp.at[pl.ds(sid * rpa + q * 8, 8)])
            plsc.subcore_barrier()
            off = aoffs[c]
            cnt = ptrs[c] - off
            trip = (cnt + KB - 1) >> 5

            @pl.loop(0, trip)
            def _(j):
                base = off + j * KB
                idxw[pl.ds(0, 16)] = iota * 0
                idxd[pl.ds(0, 16)] = iota * 0 + CH
                idxf[pl.ds(0, 16)] = iota * 0
                idxe[pl.ds(0, 16)] = iota * 0
                for q in range(1, (KB + 16) // 16):
                    o2 = base + (q - 1) * 16
                    idxw[pl.ds(q * 16, 16)] = plsc.load_gather(
                        bes, [o2 + iota])
                    dl2 = plsc.load_gather(bdl, [o2 + iota])
                    idxd[pl.ds(q * 16, 16)] = dl2
                    idxf[pl.ds(q * 16, 16)] = dl2 + c * CH
                    idxe[pl.ds(q * 16, 16)] = plsc.load_gather(
                        bei, [o2 + iota])
                c1 = pltpu.async_copy(h2_h.at[idxw], hbuf, sem1)
                c2 = pltpu.async_copy(ex_h.at[idxe], exg, sem2)
                c3 = pltpu.async_copy(d0_h.at[idxf], d0g, sem3)
                c4 = pltpu.async_copy(d1_h.at[idxf], d1g, sem4)
                c2.wait()
                c3.wait()
                c4.wait()
                for q in range(KB * 8 // 16):
                    ridx = 16 + q * 2 + (iota >> 3)
                    cdx = iota & 7
                    den = plsc.load_gather(d0g, [ridx, cdx]) + \
                        plsc.load_gather(d1g, [ridx, cdx]) + 1e-16
                    wv = plsc.load_gather(exg, [ridx, cdx]) / den
                    valid = (base + q * 2 + (iota >> 3)) < (off + cnt)
                    wv = jnp.where(valid, wv, 0.0)
                    wbuf[pl.ds(128 + q * 16, 16)] = wv
                c1.wait()
                for r in range(16, KB + 16):
                    bws = [plsc.load_gather(wbuf, [iota * 0 + (r * 8 + h)])
                           for h in range(4)]
                    rowi = iota * 0 + r
                    for kk in range(HC // 16):
                        col = kk * 16 + iota
                        hv = plsc.load_gather(hbuf, [rowi, col])
                        hv = hv * bws[(kk * 16) // (HC // 4)]
                        plsc.store_scatter(hbuf, [rowi, col], hv)
                pltpu.sync_copy(hbuf, acc_sp.at[idxd], add=True)

            plsc.subcore_barrier()
            for q in range(rpa // 8):
                pltpu.sync_copy(
                    acc_sp.at[pl.ds(sid * rpa + q * 8, 8)],
                    outp_h.at[cid].at[pl.ds(c * CH + sid * rpa + q * 8,
                                            8)])
            plsc.subcore_barrier()

    return k(es, ed, ex, den0, den1, h2)


def _post1_body(ns, concat, HC, dout,
                p0_ref, p1_ref, h2_ref, exs_ref, d0_ref, d1_ref, sk_ref,
                b_ref, hf_ref, sums_ref):
    i = pl.program_id(0)
    bm = hf_ref.shape[0]
    den = d0_ref[:, :4] + d1_ref[:, :4] + 1e-16
    ws = exs_ref[:, :4] / den
    C = HC // 4
    segs = []
    for h in range(4):
        sl = slice(h * C, (h + 1) * C)
        segs.append(p0_ref[:, sl] + p1_ref[:, sl]
                    + h2_ref[:, sl] * ws[:, h:h + 1])
    if concat:
        hf = jnp.concatenate(segs, axis=1)
    else:
        hf = 0.25 * (segs[0] + segs[1] + segs[2] + segs[3])
    hf = hf + b_ref[...] + sk_ref[...]
    hf_ref[...] = hf
    rows = lax.broadcasted_iota(jnp.int32, (bm, dout), 0) + i * bm
    valid = rows < ns
    hm = jnp.where(valid, hf, 0.0)
    s1 = jnp.sum(hm, axis=0, keepdims=True)
    s2 = jnp.sum(hm * hm, axis=0, keepdims=True)
    contrib = jnp.concatenate([s1, s2], axis=0)

    @pl.when(i == 0)
    def _():
        sums_ref[...] = jnp.zeros_like(sums_ref)

    sums_ref[...] += contrib


def _post1(p0, p1, h2, exs, d0, d1, sk, b, ns, NP, HC, dout, concat,
           bm=512):
    body = functools.partial(_post1_body, ns, concat, HC, dout)
    return pl.pallas_call(
        body,
        grid=(NP // bm,),
        in_specs=[
            pl.BlockSpec((bm, HC), lambda i: (i, 0)),
            pl.BlockSpec((bm, HC), lambda i: (i, 0)),
            pl.BlockSpec((bm, HC), lambda i: (i, 0)),
            pl.BlockSpec((bm, 8), lambda i: (i, 0)),
            pl.BlockSpec((bm, 8), lambda i: (i, 0)),
            pl.BlockSpec((bm, 8), lambda i: (i, 0)),
            pl.BlockSpec((bm, dout), lambda i: (i, 0)),
            pl.BlockSpec((1, dout), lambda i: (0, 0)),
        ],
        out_specs=[
            pl.BlockSpec((bm, dout), lambda i: (i, 0)),
            pl.BlockSpec((2, dout), lambda i: (0, 0)),
        ],
        out_shape=[
            jax.ShapeDtypeStruct((NP, dout), jnp.float32),
            jax.ShapeDtypeStruct((2, dout), jnp.float32),
        ],
    )(p0, p1, h2, exs, d0, d1, sk, b)


def _post2_body(ns, hf_ref, sums_ref, g_ref, be_ref, o_ref):
    m = sums_ref[0:1, :] / ns
    v = sums_ref[1:2, :] / ns - m * m
    xn = (hf_ref[...] - m) * lax.rsqrt(v + 1e-5) * g_ref[...] + be_ref[...]
    o_ref[...] = jnp.where(xn > 0, xn, jnp.exp(xn) - 1.0)


def _post2(hf, sums, g, be, ns, NP, dout, bm=512):
    body = functools.partial(_post2_body, float(ns))
    return pl.pallas_call(
        body,
        grid=(NP // bm,),
        in_specs=[
            pl.BlockSpec((bm, dout), lambda i: (i, 0)),
            pl.BlockSpec((2, dout), lambda i: (0, 0)),
            pl.BlockSpec((1, dout), lambda i: (0, 0)),
            pl.BlockSpec((1, dout), lambda i: (0, 0)),
        ],
        out_specs=pl.BlockSpec((bm, dout), lambda i: (i, 0)),
        out_shape=jax.ShapeDtypeStruct((NP, dout), jnp.float32),
    )(hf, sums, g, be)


def _layer(xin, ei, W, a_s, a_d, b, Wsk, bsk, g, be, H, C, concat, ns,
           E, CH, NCHK, EB):
    HC = H * C
    dout = HC if concat else C
    NP = ((ns + 511) // 512) * 512
    xt = xin[:ns]
    xp = jnp.pad(xt, ((0, NP - ns), (0, 0))) if NP != ns else xt
    eye4 = jnp.eye(4, dtype=jnp.float32)
    A_src = (eye4[:, None, :] * a_s[:, :, None]).reshape(HC, 4)
    A_dst = (eye4[:, None, :] * a_d[:, :, None]).reshape(HC, 4)
    Aall = jnp.concatenate([A_src, A_dst], axis=1)
    h2, sk, aa, exs = _pre(xp, W, Wsk, Aall, bsk.reshape(1, dout),
                           NP, HC, dout)
    es = ei[0]
    ed = ei[1]
    ex, denp = _phase_a(es, ed, aa, exs, NP)
    d0 = denp[0]
    d1 = denp[1]
    outp = _phase_b(es, ed, ex, d0, d1, h2, NP, HC, CH, NCHK, EB)
    hf, sums = _post1(outp[0][:NP], outp[1][:NP], h2, exs, d0, d1, sk,
                      b.reshape(1, dout), ns, NP, HC, dout, concat)
    return _post2(hf, sums, g.reshape(1, dout), be.reshape(1, dout),
                  ns, NP, dout)


def kernel(x, edge_index_0, edge_index_1, edge_index_2, size_0, size_1,
           size_2, W0, a_src0, a_dst0, b0, Wsk0, bsk0, g0, be0, W1, a_src1,
           a_dst1, b1, Wsk1, bsk1, g1, be1, W2, a_src2, a_dst2, b2, Wsk2,
           bsk2, g2, be2):
    h = _layer(x, edge_index_0, W0, a_src0, a_dst0, b0, Wsk0, bsk0, g0,
               be0, 4, 64, True, 40000, 500000, 4096, 10, 16176)
    h = _layer(h, edge_index_1, W1, a_src1, a_dst1, b1, Wsk1, bsk1, g1,
               be1, 4, 64, True, 10000, 160000, 4096, 3, 5376)
    h = _layer(h, edge_index_2, W2, a_src2, a_dst2, b2, Wsk2, bsk2, g2,
               be2, 4, 128, False, 2500, 40000, 2560, 1, 1664)
    return h[:2500]
